# baseline (speedup 1.0000x reference)
import functools

import jax
import jax.numpy as jnp
from jax import lax
from jax.experimental import pallas as pl
from jax.experimental.pallas import tpu as pltpu
from jax.experimental.pallas import tpu_sc as plsc

_BATCH = 16384
_NREL = 1000
_NHOI = 10000
_L = 16
_NC, _NS = 2, 16
_NW = _NC * _NS
_ROWS_PER_W = _BATCH // _NW
_R = 4
_NBLK = _ROWS_PER_W // _R
_CHUNKS = _NHOI // _L

_mesh = plsc.VectorSubcoreMesh(core_axis_name="c", subcore_axis_name="s")


@functools.partial(
    pl.kernel,
    out_type=jax.ShapeDtypeStruct((_BATCH * _NHOI,), jnp.float32),
    mesh=_mesh,
    compiler_params=pltpu.CompilerParams(needs_layout_passes=False),
    scratch_types=[
        pltpu.VMEM((_NHOI,), jnp.int32),
        pltpu.VMEM((_R * _NREL,), jnp.float32),
        pltpu.VMEM((_R * _NHOI,), jnp.float32),
    ],
)
def _gather_columns(rp_hbm, map_hbm, out_hbm, map_v, in_v, out_v):
    wid = lax.axis_index("s") * _NC + lax.axis_index("c")
    row0 = wid * _ROWS_PER_W
    pltpu.sync_copy(map_hbm, map_v)

    def block(blk, carry):
        base = row0 + blk * _R
        pltpu.sync_copy(rp_hbm.at[pl.ds(base * _NREL, _R * _NREL)], in_v)

        def chunk(j, c):
            idx = map_v[pl.ds(j * _L, _L)]
            for r in range(_R):
                flat = idx if r == 0 else idx + r * _NREL
                vals = plsc.load_gather(in_v, [flat])
                out_v[pl.ds(r * _NHOI + j * _L, _L)] = vals
            return c

        lax.fori_loop(0, _CHUNKS, chunk, 0)
        pltpu.sync_copy(out_v, out_hbm.at[pl.ds(base * _NHOI, _R * _NHOI)])
        return carry

    lax.fori_loop(0, _NBLK, block, 0)


def kernel(relation_prob, hoi_to_relation):
    rp_flat = relation_prob.reshape(-1)
    idx = hoi_to_relation.astype(jnp.int32)
    out = _gather_columns(rp_flat, idx)
    return out.reshape(_BATCH, _NHOI)


# speedup vs baseline: 1.9775x; 1.9775x over previous
import functools

import jax
import jax.numpy as jnp
from jax import lax
from jax.experimental import pallas as pl
from jax.experimental.pallas import tpu as pltpu
from jax.experimental.pallas import tpu_sc as plsc

_BATCH = 16384
_NREL = 1000
_NHOI = 10000
_L = 16
_NC, _NS = 2, 16
_NW = _NC * _NS
_ROWS_PER_W = _BATCH // _NW
_R = 4
_NBLK = _ROWS_PER_W // _R
_CHUNKS = _NHOI // _L

_mesh = plsc.VectorSubcoreMesh(core_axis_name="c", subcore_axis_name="s")


@functools.partial(
    pl.kernel,
    out_type=jax.ShapeDtypeStruct((_BATCH * _NHOI,), jnp.float32),
    mesh=_mesh,
    compiler_params=pltpu.CompilerParams(needs_layout_passes=False),
    scratch_types=[
        pltpu.VMEM((_NHOI,), jnp.int32),
        pltpu.VMEM((_R * _NREL,), jnp.float32),
        pltpu.VMEM((_R * _NREL,), jnp.float32),
        pltpu.VMEM((_R * _NHOI,), jnp.float32),
        pltpu.VMEM((_R * _NHOI,), jnp.float32),
        pltpu.SemaphoreType.DMA,
        pltpu.SemaphoreType.DMA,
        pltpu.SemaphoreType.DMA,
        pltpu.SemaphoreType.DMA,
    ],
)
def _gather_columns(rp_hbm, map_hbm, out_hbm, map_v,
                    in0, in1, out0, out1, sin0, sin1, sout0, sout1):
    wid = lax.axis_index("s") * _NC + lax.axis_index("c")
    row0 = wid * _ROWS_PER_W
    pltpu.sync_copy(map_hbm, map_v)

    ins, outs = (in0, in1), (out0, out1)
    sins, souts = (sin0, sin1), (sout0, sout1)

    def start_in(blk, b):
        base = row0 + blk * _R
        pltpu.async_copy(rp_hbm.at[pl.ds(base * _NREL, _R * _NREL)],
                         ins[b], sins[b])

    def wait_in(b):
        pltpu.make_async_copy(rp_hbm.at[pl.ds(0, _R * _NREL)],
                              ins[b], sins[b]).wait()

    def start_out(blk, b):
        base = row0 + blk * _R
        pltpu.async_copy(outs[b],
                         out_hbm.at[pl.ds(base * _NHOI, _R * _NHOI)], souts[b])

    def wait_out(b):
        pltpu.make_async_copy(outs[b],
                              out_hbm.at[pl.ds(0, _R * _NHOI)], souts[b]).wait()

    start_in(0, 0)
    start_in(1, 1)

    def super_block(g, carry):
        for b in range(2):
            blk = g * 2 + b
            wait_in(b)

            @pl.when(g > 0)
            def _():
                wait_out(b)

            @plsc.parallel_loop(0, _CHUNKS, unroll=8)
            def _chunk(j):
                idx = map_v[pl.ds(j * _L, _L)]
                for r in range(_R):
                    flat = idx if r == 0 else idx + r * _NREL
                    vals = plsc.load_gather(ins[b], [flat])
                    outs[b][pl.ds(r * _NHOI + j * _L, _L)] = vals

            start_out(blk, b)

            @pl.when(blk + 2 < _NBLK)
            def _():
                start_in(blk + 2, b)
        return carry

    lax.fori_loop(0, _NBLK // 2, super_block, 0)
    wait_out(0)
    wait_out(1)


def kernel(relation_prob, hoi_to_relation):
    rp_flat = relation_prob.reshape(-1)
    idx = hoi_to_relation.astype(jnp.int32)
    out = _gather_columns(rp_flat, idx)
    return out.reshape(_BATCH, _NHOI)


# speedup vs baseline: 2.4011x; 1.2143x over previous
import functools

import jax
import jax.numpy as jnp
from jax import lax
from jax.experimental import pallas as pl
from jax.experimental.pallas import tpu as pltpu
from jax.experimental.pallas import tpu_sc as plsc

_BATCH = 16384
_NREL = 1000
_NHOI = 10000
_L = 16
_NC, _NS = 2, 16
_NW = _NC * _NS
_CW = 128
_NCOL = _BATCH // _CW
_COL_PER_W = _NCOL // _NW
_TH = 8
_IBLK = 400
_NIB = _NHOI // _IBLK
_PAIRS = _IBLK // (2 * _TH)

_mesh = plsc.VectorSubcoreMesh(core_axis_name="c", subcore_axis_name="s")


@functools.partial(
    pl.kernel,
    out_type=jax.ShapeDtypeStruct((_NHOI, _BATCH), jnp.float32),
    mesh=_mesh,
    compiler_params=pltpu.CompilerParams(needs_layout_passes=False),
    scratch_types=[
        pltpu.VMEM((_NREL, _CW), jnp.float32),
        pltpu.VMEM((_IBLK,), jnp.int32),
        pltpu.VMEM((_IBLK,), jnp.int32),
        pltpu.VMEM((_TH, _CW), jnp.float32),
        pltpu.VMEM((_TH, _CW), jnp.float32),
        pltpu.SemaphoreType.DMA,
        pltpu.SemaphoreType.DMA,
        pltpu.SemaphoreType.DMA,
    ],
)
def _gather_rows(rp_t, map_hbm, out_t, staged, ring0, ring1, obuf0, obuf1,
                 sem_idx, sem0, sem1):
    wid = lax.axis_index("s") * _NC + lax.axis_index("c")

    obufs = (obuf0, obuf1)
    rings = (ring0, ring1)
    sems = (sem0, sem1)

    def fire_idx(ib, ring):
        pltpu.async_copy(map_hbm.at[pl.ds(ib * _IBLK, _IBLK)], ring, sem_idx)

    def wait_idx():
        pltpu.make_async_copy(map_hbm.at[pl.ds(0, _IBLK)],
                              ring0, sem_idx).wait()

    def wait_out(b):
        pltpu.make_async_copy(
            obufs[b], out_t.at[pl.ds(0, _TH), pl.ds(0, _CW)], sems[b]
        ).wait()

    def column(col_i, carry):
        bcol = (wid * _COL_PER_W + col_i) * _CW
        pltpu.sync_copy(rp_t.at[:, pl.ds(bcol, _CW)], staged)
        fire_idx(0, ring0)

        def do_block(ib, ring, next_ring):
            wait_idx()

            @pl.when(ib + 1 < _NIB)
            def _():
                fire_idx(ib + 1, next_ring)

            def pair(s, c3):
                idx16 = ring[pl.ds(s * 2 * _TH, 2 * _TH)]
                for b in range(2):
                    c = ib * 2 * _PAIRS + s * 2 + b

                    @pl.when(c >= 2)
                    def _():
                        wait_out(b)

                    for k in range(_TH):
                        r = idx16[b * _TH + k]
                        for b0 in range(0, _CW, _L):
                            obufs[b][k, pl.ds(b0, _L)] = staged[r, pl.ds(b0, _L)]
                    pltpu.async_copy(
                        obufs[b],
                        out_t.at[pl.ds(c * _TH, _TH), pl.ds(bcol, _CW)],
                        sems[b],
                    )
                return c3

            lax.fori_loop(0, _PAIRS, pair, 0)

        def superblock(sb, c2):
            do_block(2 * sb, ring0, ring1)
            do_block(2 * sb + 1, ring1, ring0)
            return c2

        lax.fori_loop(0, _NIB // 2, superblock, 0)
        do_block(_NIB - 1, ring0, ring1)
        wait_out(0)
        wait_out(1)
        return carry

    lax.fori_loop(0, _COL_PER_W, column, 0)


def kernel(relation_prob, hoi_to_relation):
    idx = hoi_to_relation.astype(jnp.int32)
    out_t = _gather_rows(relation_prob.T, idx)
    return out_t.T


# speedup vs baseline: 6.1972x; 2.5810x over previous
import functools

import jax
import jax.numpy as jnp
from jax import lax
from jax.experimental import pallas as pl
from jax.experimental.pallas import tpu as pltpu
from jax.experimental.pallas import tpu_sc as plsc

_BATCH = 16384
_NREL = 1000
_NHOI = 10000
_L = 16
_NC, _NS = 2, 16
_NW = _NC * _NS
_CW = 128
_NCOL = _BATCH // _CW
_COL_PER_W = _NCOL // _NW
_TH = 8
_IBLK = 400
_NIB = _NHOI // _IBLK
_PAIRS = _IBLK // (2 * _TH)

_mesh = plsc.VectorSubcoreMesh(core_axis_name="c", subcore_axis_name="s")


@functools.partial(
    pl.kernel,
    out_type=jax.ShapeDtypeStruct((_NHOI, _BATCH), jnp.float32),
    mesh=_mesh,
    compiler_params=pltpu.CompilerParams(needs_layout_passes=False),
    scratch_types=[
        pltpu.VMEM((_NREL, _CW), jnp.float32),
        pltpu.VMEM((_IBLK,), jnp.int32),
        pltpu.VMEM((_IBLK,), jnp.int32),
        pltpu.VMEM((_TH, _CW), jnp.float32),
        pltpu.VMEM((_TH, _CW), jnp.float32),
        pltpu.SemaphoreType.DMA,
        pltpu.SemaphoreType.DMA,
        pltpu.SemaphoreType.DMA,
    ],
)
def _gather_rows(rp_t, map_hbm, out_t, staged, ring0, ring1, obuf0, obuf1,
                 sem_idx, sem0, sem1):
    wid = lax.axis_index("s") * _NC + lax.axis_index("c")

    obufs = (obuf0, obuf1)
    rings = (ring0, ring1)
    sems = (sem0, sem1)

    def fire_idx(ib, ring):
        pltpu.async_copy(map_hbm.at[pl.ds(ib * _IBLK, _IBLK)], ring, sem_idx)

    def wait_idx():
        pltpu.make_async_copy(map_hbm.at[pl.ds(0, _IBLK)],
                              ring0, sem_idx).wait()

    def wait_out(b):
        pltpu.make_async_copy(
            obufs[b], out_t.at[pl.ds(0, _TH), pl.ds(0, _CW)], sems[b]
        ).wait()

    def column(col_i, carry):
        bcol = (wid * _COL_PER_W + col_i) * _CW
        pltpu.sync_copy(rp_t.at[:, pl.ds(bcol, _CW)], staged)
        fire_idx(0, ring0)

        def do_block(ib, ring, next_ring):
            wait_idx()

            @pl.when(ib + 1 < _NIB)
            def _():
                fire_idx(ib + 1, next_ring)

            def pair(s, c3):
                idx16 = ring[pl.ds(s * 2 * _TH, 2 * _TH)]
                for b in range(2):
                    c = ib * 2 * _PAIRS + s * 2 + b

                    @pl.when(c >= 2)
                    def _():
                        wait_out(b)

                    prev = None
                    for k in range(_TH):
                        r = idx16[b * _TH + k]
                        cur = []
                        for i, b0 in enumerate(range(0, _CW, _L)):
                            cur.append(staged[r, pl.ds(b0, _L)])
                            if prev is not None:
                                obufs[b][k - 1, pl.ds(i * _L, _L)] = prev[i]
                        prev = cur
                    for i, v in enumerate(prev):
                        obufs[b][_TH - 1, pl.ds(i * _L, _L)] = v
                    pltpu.async_copy(
                        obufs[b],
                        out_t.at[pl.ds(c * _TH, _TH), pl.ds(bcol, _CW)],
                        sems[b],
                    )
                return c3

            lax.fori_loop(0, _PAIRS, pair, 0)

        def superblock(sb, c2):
            do_block(2 * sb, ring0, ring1)
            do_block(2 * sb + 1, ring1, ring0)
            return c2

        lax.fori_loop(0, _NIB // 2, superblock, 0)
        do_block(_NIB - 1, ring0, ring1)
        wait_out(0)
        wait_out(1)
        return carry

    lax.fori_loop(0, _COL_PER_W, column, 0)


def kernel(relation_prob, hoi_to_relation):
    idx = hoi_to_relation.astype(jnp.int32)
    out_t = _gather_rows(relation_prob.T, idx)
    return out_t.T


# speedup vs baseline: 6.6675x; 1.0759x over previous
import functools

import jax
import jax.numpy as jnp
from jax import lax
from jax.experimental import pallas as pl
from jax.experimental.pallas import tpu as pltpu
from jax.experimental.pallas import tpu_sc as plsc

_BATCH = 16384
_NREL = 1000
_NHOI = 10000
_L = 16
_NC, _NS = 2, 16
_NW = _NC * _NS
_CW = 128
_NCOL = _BATCH // _CW
_COL_PER_W = _NCOL // _NW
_TH = 8
_IBLK = 400
_NIB = _NHOI // _IBLK
_PAIRS = _IBLK // (2 * _TH)

_mesh = plsc.VectorSubcoreMesh(core_axis_name="c", subcore_axis_name="s")


@functools.partial(
    pl.kernel,
    out_type=jax.ShapeDtypeStruct((_NHOI, _BATCH), jnp.float32),
    mesh=_mesh,
    compiler_params=pltpu.CompilerParams(needs_layout_passes=False),
    scratch_types=[
        pltpu.VMEM((_NREL, _CW), jnp.float32),
        pltpu.VMEM((_IBLK,), jnp.int32),
        pltpu.VMEM((_IBLK,), jnp.int32),
        pltpu.VMEM((_TH, _CW), jnp.float32),
        pltpu.VMEM((_TH, _CW), jnp.float32),
        pltpu.SemaphoreType.DMA,
        pltpu.SemaphoreType.DMA,
        pltpu.SemaphoreType.DMA,
    ],
)
def _gather_rows(rp_t, map_hbm, out_t, staged, ring0, ring1, obuf0, obuf1,
                 sem_idx, sem0, sem1):
    wid = lax.axis_index("s") * _NC + lax.axis_index("c")

    obufs = (obuf0, obuf1)
    rings = (ring0, ring1)
    sems = (sem0, sem1)

    def fire_idx(ib, ring):
        pltpu.async_copy(map_hbm.at[pl.ds(ib * _IBLK, _IBLK)], ring, sem_idx)

    def wait_idx():
        pltpu.make_async_copy(map_hbm.at[pl.ds(0, _IBLK)],
                              ring0, sem_idx).wait()

    def wait_out(b):
        pltpu.make_async_copy(
            obufs[b], out_t.at[pl.ds(0, _TH), pl.ds(0, _CW)], sems[b]
        ).wait()

    def column(col_i, carry):
        bcol = (wid * _COL_PER_W + col_i) * _CW
        pltpu.sync_copy(rp_t.at[:, pl.ds(bcol, _CW)], staged)
        fire_idx(0, ring0)

        def do_block(ib, ring, next_ring):
            wait_idx()

            @pl.when(ib + 1 < _NIB)
            def _():
                fire_idx(ib + 1, next_ring)

            def pair(s, c3):
                idx16 = ring[pl.ds(s * 2 * _TH, 2 * _TH)]
                rs = [idx16[j] for j in range(2 * _TH)]
                for b in range(2):
                    c = ib * 2 * _PAIRS + s * 2 + b

                    @pl.when(c >= 2)
                    def _():
                        wait_out(b)

                    prev = None
                    for k in range(_TH):
                        r = rs[b * _TH + k]
                        cur = []
                        for i, b0 in enumerate(range(0, _CW, _L)):
                            cur.append(staged[r, pl.ds(b0, _L)])
                            if prev is not None:
                                obufs[b][k - 1, pl.ds(i * _L, _L)] = prev[i]
                        prev = cur
                    for i, v in enumerate(prev):
                        obufs[b][_TH - 1, pl.ds(i * _L, _L)] = v
                    pltpu.async_copy(
                        obufs[b],
                        out_t.at[pl.ds(c * _TH, _TH), pl.ds(bcol, _CW)],
                        sems[b],
                    )
                return c3

            lax.fori_loop(0, _PAIRS, pair, 0)

        def superblock(sb, c2):
            do_block(2 * sb, ring0, ring1)
            do_block(2 * sb + 1, ring1, ring0)
            return c2

        lax.fori_loop(0, _NIB // 2, superblock, 0)
        do_block(_NIB - 1, ring0, ring1)
        wait_out(0)
        wait_out(1)
        return carry

    lax.fori_loop(0, _COL_PER_W, column, 0)


def kernel(relation_prob, hoi_to_relation):
    idx = hoi_to_relation.astype(jnp.int32)
    out_t = _gather_rows(relation_prob.T, idx)
    return out_t.T
